# baseline (speedup 1.0000x reference)
import math

import jax
import jax.numpy as jnp
from jax import lax
from jax.experimental import pallas as pl
from jax.experimental.pallas import tpu as pltpu, tpu_sc as plsc

N_TOKEN = 1000000
D_MODEL = 64
SCALE = math.sqrt(D_MODEL)

NC = 2
NS = 16
NW = NC * NS
LANES = 16

IDX_W = 128
K = 4
CHUNK = K * IDX_W


def _sc_embed(idx2d, table):
    n_idx_rows = idx2d.shape[0]
    rows_per_w = n_idx_rows // NW
    n_chunks = rows_per_w // K
    b = n_idx_rows * IDX_W

    mesh = plsc.VectorSubcoreMesh(
        core_axis_name="c", subcore_axis_name="s",
        num_cores=NC, num_subcores=NS,
    )

    def body(idx_hbm, table_hbm, out_hbm, idx_v, rows_v, sem):
        wid = lax.axis_index("s") * NC + lax.axis_index("c")
        row0 = wid * rows_per_w

        def chunk_body(i, carry):
            irow = row0 + i * K
            pltpu.sync_copy(idx_hbm.at[pl.ds(irow, K)], idx_v)
            copies = [
                pltpu.async_copy(
                    table_hbm.at[idx_v.at[j]],
                    rows_v.at[pl.ds(j * IDX_W, IDX_W)],
                    sem,
                )
                for j in range(K)
            ]
            for c in copies:
                c.wait()

            def scale_row(r, carry2):
                for j in range(D_MODEL // LANES):
                    sl = pl.ds(j * LANES, LANES)
                    rows_v[r, sl] = rows_v[r, sl] * SCALE
                return carry2

            lax.fori_loop(0, CHUNK, scale_row, 0, unroll=2)
            pltpu.sync_copy(rows_v, out_hbm.at[pl.ds(irow * IDX_W, CHUNK)])
            return carry

        lax.fori_loop(0, n_chunks, chunk_body, 0)

    run = pl.kernel(
        body,
        out_type=jax.ShapeDtypeStruct((b, D_MODEL), jnp.float32),
        mesh=mesh,
        compiler_params=pltpu.CompilerParams(use_tc_tiling_on_sc=False),
        scratch_types=[
            pltpu.VMEM((K, IDX_W), jnp.int32),
            pltpu.VMEM((CHUNK, D_MODEL), jnp.float32),
            pltpu.SemaphoreType.DMA,
        ],
    )
    return run(idx2d, table)


def kernel(x, lut_weight):
    b0, b1 = x.shape
    idx2d = x.reshape(-1, IDX_W)
    out = _sc_embed(idx2d, lut_weight)
    return out.reshape(b0, b1, D_MODEL)


# speedup vs baseline: 1.0876x; 1.0876x over previous
import math

import jax
import jax.numpy as jnp
from jax import lax
from jax.experimental import pallas as pl
from jax.experimental.pallas import tpu as pltpu, tpu_sc as plsc

N_TOKEN = 1000000
D_MODEL = 64
SCALE = math.sqrt(D_MODEL)

NC = 2
NS = 16
NW = NC * NS
LANES = 16

IDX_W = 128
K = 4
CHUNK = K * IDX_W


def _sc_embed(idx2d, table):
    n_idx_rows = idx2d.shape[0]
    rows_per_w = n_idx_rows // NW
    n_chunks = rows_per_w // K
    b = n_idx_rows * IDX_W

    mesh = plsc.VectorSubcoreMesh(
        core_axis_name="c", subcore_axis_name="s",
        num_cores=NC, num_subcores=NS,
    )

    def body(idx_hbm, table_hbm, out_hbm, idx_v, rows0, rows1,
             sem_in0, sem_in1, sem_out0, sem_out1):
        wid = lax.axis_index("s") * NC + lax.axis_index("c")
        row0 = wid * rows_per_w
        pltpu.sync_copy(idx_hbm.at[pl.ds(row0, rows_per_w)], idx_v)

        rows = (rows0, rows1)
        sem_in = (sem_in0, sem_in1)
        sem_out = (sem_out0, sem_out1)

        def fire(g, p):
            for j in range(K):
                pltpu.async_copy(
                    table_hbm.at[idx_v.at[g * K + j]],
                    rows[p].at[pl.ds(j * IDX_W, IDX_W)],
                    sem_in[p],
                )

        def drain(p):
            pltpu.make_async_copy(
                table_hbm.at[pl.ds(0, CHUNK)], rows[p], sem_in[p]
            ).wait()

        def scale(p):
            rp = rows[p]

            @plsc.parallel_loop(0, CHUNK, step=1, unroll=8)
            def _(r):
                for j in range(D_MODEL // LANES):
                    sl = pl.ds(j * LANES, LANES)
                    rp[r, sl] = rp[r, sl] * SCALE

        def out_slice(g):
            return out_hbm.at[pl.ds((row0 + g * K) * IDX_W, CHUNK)]

        def fire_out(g, p):
            pltpu.async_copy(rows[p], out_slice(g), sem_out[p])

        def wait_out(p):
            pltpu.make_async_copy(
                rows[p], out_hbm.at[pl.ds(0, CHUNK)], sem_out[p]
            ).wait()

        fire(0, 0)
        fire(1, 1)
        drain(0)
        scale(0)
        fire_out(0, 0)

        def outer(i, carry):
            for bb in (0, 1):
                g = 2 * i + 1 + bb
                p = (1 + bb) & 1
                wait_out(p ^ 1)
                fire(g + 1, p ^ 1)
                drain(p)
                scale(p)
                fire_out(g, p)
            return carry

        lax.fori_loop(0, (n_chunks - 2) // 2, outer, 0)

        drain(1)
        scale(1)
        fire_out(n_chunks - 1, 1)
        wait_out(0)
        wait_out(1)

    run = pl.kernel(
        body,
        out_type=jax.ShapeDtypeStruct((b, D_MODEL), jnp.float32),
        mesh=mesh,
        compiler_params=pltpu.CompilerParams(use_tc_tiling_on_sc=False),
        scratch_types=[
            pltpu.VMEM((rows_per_w, IDX_W), jnp.int32),
            pltpu.VMEM((CHUNK, D_MODEL), jnp.float32),
            pltpu.VMEM((CHUNK, D_MODEL), jnp.float32),
            pltpu.SemaphoreType.DMA,
            pltpu.SemaphoreType.DMA,
            pltpu.SemaphoreType.DMA,
            pltpu.SemaphoreType.DMA,
        ],
    )
    return run(idx2d, table)


def kernel(x, lut_weight):
    b0, b1 = x.shape
    idx2d = x.reshape(-1, IDX_W)
    out = _sc_embed(idx2d, lut_weight)
    return out.reshape(b0, b1, D_MODEL)


# speedup vs baseline: 1.3293x; 1.2223x over previous
import math

import jax
import jax.numpy as jnp
from jax import lax
from jax.experimental import pallas as pl
from jax.experimental.pallas import tpu as pltpu, tpu_sc as plsc

N_TOKEN = 1000000
D_MODEL = 64
SCALE = math.sqrt(D_MODEL)

NC = 2
NS = 16
NW = NC * NS
LANES = 16

ROW_W = 128
IDX_W = 128
K = 2
CHUNK = K * IDX_W


def _sc_embed(idx2d, table_p):
    n_idx_rows = idx2d.shape[0]
    rows_per_w = n_idx_rows // NW
    n_chunks = rows_per_w // K
    b = n_idx_rows * IDX_W

    mesh = plsc.VectorSubcoreMesh(
        core_axis_name="c", subcore_axis_name="s",
        num_cores=NC, num_subcores=NS,
    )

    def body(idx_hbm, table_hbm, out_hbm, idx_v, rows0, rows1,
             sem_in0, sem_in1, sem_out0, sem_out1):
        wid = lax.axis_index("s") * NC + lax.axis_index("c")
        row0 = wid * rows_per_w
        pltpu.sync_copy(idx_hbm.at[pl.ds(row0, rows_per_w)], idx_v)

        rows = (rows0, rows1)
        sem_in = (sem_in0, sem_in1)
        sem_out = (sem_out0, sem_out1)

        def fire(g, p):
            for j in range(K):
                pltpu.async_copy(
                    table_hbm.at[idx_v.at[g * K + j]],
                    rows[p].at[pl.ds(j * IDX_W, IDX_W)],
                    sem_in[p],
                )

        def drain(p):
            pltpu.make_async_copy(
                table_hbm.at[pl.ds(0, CHUNK)], rows[p], sem_in[p]
            ).wait()

        def scale(p):
            rp = rows[p]

            @plsc.parallel_loop(0, CHUNK, step=1, unroll=8)
            def _(r):
                for j in range(D_MODEL // LANES):
                    sl = pl.ds(j * LANES, LANES)
                    rp[r, sl] = rp[r, sl] * SCALE

        def out_slice(g):
            return out_hbm.at[pl.ds((row0 + g * K) * IDX_W, CHUNK)]

        def fire_out(g, p):
            pltpu.async_copy(rows[p], out_slice(g), sem_out[p])

        def wait_out(p):
            pltpu.make_async_copy(
                rows[p], out_hbm.at[pl.ds(0, CHUNK)], sem_out[p]
            ).wait()

        fire(0, 0)
        fire(1, 1)
        drain(0)
        scale(0)
        fire_out(0, 0)

        def outer(i, carry):
            for bb in (0, 1):
                g = 2 * i + 1 + bb
                p = (1 + bb) & 1
                wait_out(p ^ 1)
                fire(g + 1, p ^ 1)
                drain(p)
                scale(p)
                fire_out(g, p)
            return carry

        lax.fori_loop(0, (n_chunks - 2) // 2, outer, 0)

        drain(1)
        scale(1)
        fire_out(n_chunks - 1, 1)
        wait_out(0)
        wait_out(1)

    run = pl.kernel(
        body,
        out_type=jax.ShapeDtypeStruct((b, ROW_W), jnp.float32),
        mesh=mesh,
        compiler_params=pltpu.CompilerParams(use_tc_tiling_on_sc=True),
        scratch_types=[
            pltpu.VMEM((rows_per_w, IDX_W), jnp.int32),
            pltpu.VMEM((CHUNK, ROW_W), jnp.float32),
            pltpu.VMEM((CHUNK, ROW_W), jnp.float32),
            pltpu.SemaphoreType.DMA,
            pltpu.SemaphoreType.DMA,
            pltpu.SemaphoreType.DMA,
            pltpu.SemaphoreType.DMA,
        ],
    )
    return run(idx2d, table_p)


def kernel(x, lut_weight):
    b0, b1 = x.shape
    idx2d = x.reshape(-1, IDX_W)
    table_p = jnp.pad(lut_weight, ((0, 0), (0, ROW_W - D_MODEL)))
    o2 = _sc_embed(idx2d, table_p)
    return o2.reshape(b0, b1, ROW_W)[:, :, :D_MODEL]
